# baseline (speedup 1.0000x reference)
import functools

import jax
import jax.numpy as jnp
from jax import lax
from jax.experimental import pallas as pl
from jax.experimental.pallas import tpu as pltpu
from jax.experimental.pallas import tpu_sc as plsc

D = 64
B = 16384
NC = 2
NS = 16
NW = NC * NS
BPW = B // NW


def _gather_sc(idx2, table3):
    mesh = plsc.VectorSubcoreMesh(core_axis_name="c", subcore_axis_name="s")

    @functools.partial(
        pl.kernel,
        mesh=mesh,
        out_type=jax.ShapeDtypeStruct((B // 2, 2 * D), jnp.float32),
        scratch_types=[
            pltpu.VMEM((BPW,), jnp.int32),
            pltpu.VMEM((BPW // 2, 2 * D), jnp.float32),
            pltpu.SemaphoreType.DMA,
        ],
        compiler_params=pltpu.CompilerParams(use_tc_tiling_on_sc=False),
    )
    def k(idx_hbm, table_hbm, out_hbm, idx_v, rows, semg):
        wid = lax.axis_index("s") * NC + lax.axis_index("c")
        base = wid * BPW
        pltpu.sync_copy(idx_hbm.at[wid], idx_v)
        copies = []
        for g in range(BPW // 16):
            rv = idx_v[pl.ds(g * 16, 16)]
            tv = lax.shift_right_logical(rv, 3)
            sv = rv & 7
            for l in range(16):
                i = g * 16 + l
                copies.append(
                    pltpu.async_copy(
                        table_hbm.at[tv[l], sv[l]],
                        rows.at[i // 2, pl.ds((i % 2) * D, D)], semg))
        for c in copies:
            c.wait()
        pltpu.sync_copy(rows, out_hbm.at[pl.ds(base // 2, BPW // 2)])

    return k(idx2, table3)


def _bn_tc(e2, w, b):
    def body(e_ref, w_ref, b_ref, o_ref):
        x = e_ref[...]
        s = jnp.sum(x, axis=0, keepdims=True)
        sq = jnp.sum(x * x, axis=0, keepdims=True)
        m = (s[:, :D] + s[:, D:]) * (1.0 / B)
        msq = (sq[:, :D] + sq[:, D:]) * (1.0 / B)
        var = msq - m * m
        scale = w_ref[...] * lax.rsqrt(var + 1e-5)
        shift = b_ref[...] - m * scale
        scale2 = jnp.concatenate([scale, scale], axis=1)
        shift2 = jnp.concatenate([shift, shift], axis=1)
        o_ref[...] = x * scale2 + shift2

    return pl.pallas_call(
        body,
        out_shape=jax.ShapeDtypeStruct((B // 2, 2 * D), jnp.float32),
    )(e2, w.reshape(1, D), b.reshape(1, D))


@jax.jit
def kernel(indices, embed_weight, bn_weight, bn_bias):
    idx2 = indices.astype(jnp.int32).reshape(NW, BPW)
    table3 = embed_weight.reshape(-1, 8, D)
    e2 = _gather_sc(idx2, table3)
    return _bn_tc(e2, bn_weight, bn_bias).reshape(B, D)


# speedup vs baseline: 2.4079x; 2.4079x over previous
import functools

import jax
import jax.numpy as jnp
from jax import lax
from jax.experimental import pallas as pl
from jax.experimental.pallas import tpu as pltpu
from jax.experimental.pallas import tpu_sc as plsc

D = 64
B = 16384
NC = 2
NS = 16
NW = NC * NS
BPW = B // NW


def _gather_sc(idx1, table3):
    mesh = plsc.VectorSubcoreMesh(core_axis_name="c", subcore_axis_name="s")

    @functools.partial(
        pl.kernel,
        mesh=mesh,
        out_type=jax.ShapeDtypeStruct((B, D), jnp.float32),
        scratch_types=[
            pltpu.VMEM((BPW,), jnp.int32),
            pltpu.VMEM((BPW, D), jnp.float32),
            pltpu.SemaphoreType.DMA,
        ],
    )
    def k(idx_hbm, table_hbm, out_hbm, idx_v, rows, semg):
        wid = lax.axis_index("s") * NC + lax.axis_index("c")
        base = wid * BPW
        pltpu.sync_copy(idx_hbm.at[pl.ds(base, BPW)], idx_v)
        copies = []
        for g in range(BPW // 16):
            rv = idx_v[pl.ds(g * 16, 16)]
            tv = lax.shift_right_logical(rv, 3)
            sv = rv & 7
            for l in range(16):
                copies.append(
                    pltpu.async_copy(
                        table_hbm.at[tv[l], sv[l]],
                        rows.at[g * 16 + l], semg))
        for c in copies:
            c.wait()
        pltpu.sync_copy(rows, out_hbm.at[pl.ds(base, BPW)])

    return k(idx1, table3)


def _bn_tc(e, w, b):
    def body(e_ref, w_ref, b_ref, o_ref):
        x = e_ref[...]
        mean = jnp.mean(x, axis=0, keepdims=True)
        xc = x - mean
        var = jnp.mean(xc * xc, axis=0, keepdims=True)
        inv = lax.rsqrt(var + 1e-5)
        o_ref[...] = xc * (inv * w_ref[...]) + b_ref[...]

    return pl.pallas_call(
        body,
        out_shape=jax.ShapeDtypeStruct((B, D), jnp.float32),
    )(e, w.reshape(1, D), b.reshape(1, D))


@jax.jit
def kernel(indices, embed_weight, bn_weight, bn_bias):
    table3 = embed_weight.reshape(-1, 8, D)
    e = _gather_sc(indices.astype(jnp.int32), table3)
    return _bn_tc(e, bn_weight, bn_bias)
